# baseline (speedup 1.0000x reference)
import functools

import jax
import jax.numpy as jnp
from jax import lax
from jax.experimental import pallas as pl
from jax.experimental.pallas import tpu as pltpu
from jax.experimental.pallas import tpu_sc as plsc

B = 4096
NSF = 26
NDF = 13
ROW = NSF + NDF
V = 100000
NC = 2
NSUB = 16
NW = NC * NSUB
TB = B // NW
L = 16
NCH = TB // L

_mesh = plsc.VectorSubcoreMesh(
    core_axis_name="c", subcore_axis_name="s", num_cores=NC, num_subcores=NSUB
)


@functools.partial(
    pl.kernel,
    out_type=jax.ShapeDtypeStruct((B,), jnp.float32),
    mesh=_mesh,
    scratch_types=[
        pltpu.VMEM_SHARED((NSUB * TB * ROW,), jnp.float32),
        pltpu.VMEM((ROW, TB), jnp.int32),
        pltpu.VMEM((ROW, TB), jnp.float32),
        pltpu.VMEM((NDF, L), jnp.float32),
        pltpu.VMEM((NSF, TB), jnp.int32),
        pltpu.VMEM((NSF, TB), jnp.float32),
        pltpu.VMEM((TB,), jnp.float32),
        pltpu.SemaphoreType.DMA,
        pltpu.SemaphoreType.DMA,
        pltpu.SemaphoreType.DMA,
    ],
)
def _linear_part(x_hbm, w_hbm, tbl_hbm, out_hbm,
                 x_s, xi_v, xc_v, w_v, idx_v, emb_v, acc_v,
                 sem_x, sem_c, sem_t):
    sid = lax.axis_index("s")
    wid = sid * NC + lax.axis_index("c")
    base = wid * TB

    x_copy = pltpu.async_copy(
        x_hbm.at[pl.ds(base * ROW, TB * ROW)],
        x_s.at[pl.ds(sid * TB * ROW, TB * ROW)], sem_x
    )
    pltpu.sync_copy(w_hbm, w_v)

    row_off = lax.iota(jnp.int32, L) * ROW
    sbase = sid * TB * ROW
    for c in range(ROW):
        for j in range(NCH):
            xi_v[c, pl.ds(j * L, L)] = row_off + (sbase + j * L * ROW + c)

    x_copy.wait()

    col_copies = [
        pltpu.async_copy(x_s.at[xi_v.at[c]], xc_v.at[c], sem_c)
        for c in range(ROW)
    ]

    tbl_copies = []
    for f in range(NSF):
        col_copies[f].wait()
        for j in range(NCH):
            sl = pl.ds(j * L, L)
            idx_v[f, sl] = xc_v[f, sl].astype(jnp.int32) + f * V
        tbl_copies.append(
            pltpu.async_copy(tbl_hbm.at[idx_v.at[f]], emb_v.at[f], sem_t)
        )

    for c in col_copies[NSF:]:
        c.wait()
    full = pl.ds(0, L)
    wvec = [w_v[d, full] for d in range(NDF)]
    for j in range(NCH):
        sl = pl.ds(j * L, L)
        a = None
        for d in range(NDF):
            xv = xc_v[NSF + d, sl]
            a = xv * wvec[d] if a is None else a + xv * wvec[d]
        acc_v[sl] = a

    for f in range(NSF):
        tbl_copies[f].wait()
        for j in range(NCH):
            sl = pl.ds(j * L, L)
            acc_v[sl] = acc_v[sl] + emb_v[f, sl]

    pltpu.sync_copy(acc_v, out_hbm.at[pl.ds(base, TB)])


@jax.jit
def _run(X, table, W_dense):
    wb = jnp.broadcast_to(W_dense, (NDF, L))
    out = _linear_part(X.reshape(-1), wb, table.reshape(-1))
    return out.reshape(B, 1)


def kernel(X, table, W_dense, sparse_col_idx, dense_col_idx):
    return _run(X, table, W_dense)
